# baseline (device time: 64707 ns/iter reference)
import jax
import jax.numpy as jnp
from jax import lax
from jax.experimental import pallas as pl
from jax.experimental.pallas import tpu as pltpu

N_DEV = 8
B = 2
SQ = 128
SKV = 1024
HQ = 4
DH = 64
D_MODEL = 512
D_QK = HQ * DH
KV_PER = SKV // N_DEV


def kernel(x, Wq, K_ext, V_ext, Wo):
    k_shard = K_ext.reshape(B, KV_PER, D_QK)
    v_shard = V_ext.reshape(B, KV_PER, D_QK)

    def body(x_ref, wq_ref, k_ref, v_ref, wo_ref, out_ref,
             kfull, vfull, ksend, krecv, vsend, vrecv):
        my = lax.axis_index("i")
        left = lax.rem(my + N_DEV - 1, N_DEV)
        right = lax.rem(my + 1, N_DEV)

        barrier_sem = pltpu.get_barrier_semaphore()
        for nbr in (left, right):
            pl.semaphore_signal(
                barrier_sem, inc=1,
                device_id=(nbr,), device_id_type=pl.DeviceIdType.MESH,
            )
        pl.semaphore_wait(barrier_sem, 2)

        kfull[pl.ds(my, 1)] = k_ref[...][None]
        vfull[pl.ds(my, 1)] = v_ref[...][None]

        for h in range(N_DEV - 1):
            o = lax.rem(my - h + N_DEV, N_DEV)
            rk = pltpu.make_async_remote_copy(
                src_ref=kfull.at[o], dst_ref=kfull.at[o],
                send_sem=ksend.at[h], recv_sem=krecv.at[h],
                device_id=(right,), device_id_type=pl.DeviceIdType.MESH,
            )
            rv = pltpu.make_async_remote_copy(
                src_ref=vfull.at[o], dst_ref=vfull.at[o],
                send_sem=vsend.at[h], recv_sem=vrecv.at[h],
                device_id=(right,), device_id_type=pl.DeviceIdType.MESH,
            )
            rk.start()
            rv.start()
            rk.wait()
            rv.wait()

        qi = lax.broadcasted_iota(jnp.int32, (SQ, SKV), 0)
        ki = lax.broadcasted_iota(jnp.int32, (SQ, SKV), 1)
        mask = (jnp.abs(qi - ki) <= 128) | (ki < 32) | (qi < 32)

        for b in range(B):
            q_b = jnp.dot(x_ref[b], wq_ref[...],
                          preferred_element_type=jnp.float32)
            kb = jnp.concatenate([kfull[o, b] for o in range(N_DEV)], axis=0)
            vb = jnp.concatenate([vfull[o, b] for o in range(N_DEV)], axis=0)
            ctx_heads = []
            for hh in range(HQ):
                sl = slice(hh * DH, (hh + 1) * DH)
                scores = lax.dot_general(
                    q_b[:, sl], kb[:, sl],
                    (((1,), (1,)), ((), ())),
                    preferred_element_type=jnp.float32,
                )
                s = jnp.where(mask, scores * 0.125, -1e9)
                m = jnp.max(s, axis=1, keepdims=True)
                w = jnp.exp(s - m)
                w = w / jnp.sum(w, axis=1, keepdims=True)
                ctx_heads.append(
                    jnp.dot(w, vb[:, sl], preferred_element_type=jnp.float32)
                )
            ctx = jnp.concatenate(ctx_heads, axis=1)
            out_ref[b] = jnp.dot(ctx, wo_ref[...],
                                 preferred_element_type=jnp.float32)

    return pl.pallas_call(
        body,
        out_shape=jax.ShapeDtypeStruct((B, SQ, D_MODEL), jnp.float32),
        in_specs=[pl.BlockSpec(memory_space=pltpu.VMEM)] * 5,
        out_specs=pl.BlockSpec(memory_space=pltpu.VMEM),
        scratch_shapes=[
            pltpu.VMEM((N_DEV, B, KV_PER, D_QK), jnp.float32),
            pltpu.VMEM((N_DEV, B, KV_PER, D_QK), jnp.float32),
            pltpu.SemaphoreType.DMA((N_DEV,)),
            pltpu.SemaphoreType.DMA((N_DEV,)),
            pltpu.SemaphoreType.DMA((N_DEV,)),
            pltpu.SemaphoreType.DMA((N_DEV,)),
        ],
        compiler_params=pltpu.CompilerParams(collective_id=0),
    )(x, Wq, k_shard, v_shard, Wo)


# device time: 33708 ns/iter; 1.9196x vs baseline; 1.9196x over previous
import jax
import jax.numpy as jnp
from jax import lax
from jax.experimental import pallas as pl
from jax.experimental.pallas import tpu as pltpu

N_DEV = 8
B = 2
SQ = 128
SKV = 1024
HQ = 4
DH = 64
D_MODEL = 512
D_QK = HQ * DH
KV_PER = SKV // N_DEV
NEG = -1e9


def kernel(x, Wq, K_ext, V_ext, Wo):
    k_shard = K_ext.reshape(B, KV_PER, D_QK)
    v_shard = V_ext.reshape(B, KV_PER, D_QK)

    def body(x_ref, wq_ref, k_ref, v_ref, wo_ref, out_ref,
             uall, sall, usend, urecv, ssend, srecv):
        my = lax.axis_index("i")

        qi = lax.broadcasted_iota(jnp.int32, (SQ, KV_PER), 0)
        kj = lax.broadcasted_iota(jnp.int32, (SQ, KV_PER), 1) + my * KV_PER
        mask = (jnp.abs(qi - kj) <= 128) | (kj < 32) | (qi < 32)

        for b in range(B):
            q_b = jnp.dot(x_ref[b], wq_ref[...],
                          preferred_element_type=jnp.float32)
            stats = []
            for hh in range(HQ):
                sl = slice(hh * DH, (hh + 1) * DH)
                scores = lax.dot_general(
                    q_b[:, sl], k_ref[b][:, sl],
                    (((1,), (1,)), ((), ())),
                    preferred_element_type=jnp.float32,
                )
                s = jnp.where(mask, scores * 0.125, NEG)
                m = jnp.max(s, axis=1, keepdims=True)
                p = jnp.exp(s - m)
                l = jnp.sum(p, axis=1, keepdims=True)
                u = jnp.dot(p, v_ref[b][:, sl],
                            preferred_element_type=jnp.float32)
                uall[pl.ds(my, 1), b, :, sl] = u[None]
                stats.extend([m, l])
            sall[pl.ds(my, 1), b] = jnp.concatenate(
                [stats[2 * hh] for hh in range(HQ)]
                + [stats[2 * hh + 1] for hh in range(HQ)], axis=1)[None]

        barrier_sem = pltpu.get_barrier_semaphore()
        for nbr in range(N_DEV):
            @pl.when(nbr != my)
            def _():
                pl.semaphore_signal(
                    barrier_sem, inc=1,
                    device_id=(nbr,), device_id_type=pl.DeviceIdType.MESH,
                )
        pl.semaphore_wait(barrier_sem, N_DEV - 1)

        for peer in range(N_DEV):
            @pl.when(peer != my)
            def _():
                pltpu.make_async_remote_copy(
                    src_ref=uall.at[my], dst_ref=uall.at[my],
                    send_sem=usend.at[peer], recv_sem=urecv.at[my],
                    device_id=(peer,), device_id_type=pl.DeviceIdType.MESH,
                ).start()
                pltpu.make_async_remote_copy(
                    src_ref=sall.at[my], dst_ref=sall.at[my],
                    send_sem=ssend.at[peer], recv_sem=srecv.at[my],
                    device_id=(peer,), device_id_type=pl.DeviceIdType.MESH,
                ).start()

        for o in range(N_DEV):
            @pl.when(o != my)
            def _():
                pltpu.make_async_remote_copy(
                    src_ref=uall.at[o], dst_ref=uall.at[o],
                    send_sem=usend.at[o], recv_sem=urecv.at[o],
                    device_id=(my,), device_id_type=pl.DeviceIdType.MESH,
                ).wait_recv()
                pltpu.make_async_remote_copy(
                    src_ref=sall.at[o], dst_ref=sall.at[o],
                    send_sem=ssend.at[o], recv_sem=srecv.at[o],
                    device_id=(my,), device_id_type=pl.DeviceIdType.MESH,
                ).wait_recv()

        for b in range(B):
            ctx_heads = []
            for hh in range(HQ):
                sl = slice(hh * DH, (hh + 1) * DH)
                ms = [sall[o, b][:, hh:hh + 1] for o in range(N_DEV)]
                ls = [sall[o, b][:, HQ + hh:HQ + hh + 1] for o in range(N_DEV)]
                M = ms[0]
                for o in range(1, N_DEV):
                    M = jnp.maximum(M, ms[o])
                ws = [jnp.exp(ms[o] - M) for o in range(N_DEV)]
                L = ws[0] * ls[0]
                acc = ws[0] * uall[0, b][:, sl]
                for o in range(1, N_DEV):
                    L = L + ws[o] * ls[o]
                    acc = acc + ws[o] * uall[o, b][:, sl]
                ctx_heads.append(acc / L)
            ctx = jnp.concatenate(ctx_heads, axis=1)
            out_ref[b] = jnp.dot(ctx, wo_ref[...],
                                 preferred_element_type=jnp.float32)

        for peer in range(N_DEV):
            @pl.when(peer != my)
            def _():
                pltpu.make_async_remote_copy(
                    src_ref=uall.at[my], dst_ref=uall.at[my],
                    send_sem=usend.at[peer], recv_sem=urecv.at[my],
                    device_id=(peer,), device_id_type=pl.DeviceIdType.MESH,
                ).wait_send()
                pltpu.make_async_remote_copy(
                    src_ref=sall.at[my], dst_ref=sall.at[my],
                    send_sem=ssend.at[peer], recv_sem=srecv.at[my],
                    device_id=(peer,), device_id_type=pl.DeviceIdType.MESH,
                ).wait_send()

    return pl.pallas_call(
        body,
        out_shape=jax.ShapeDtypeStruct((B, SQ, D_MODEL), jnp.float32),
        in_specs=[pl.BlockSpec(memory_space=pltpu.VMEM)] * 5,
        out_specs=pl.BlockSpec(memory_space=pltpu.VMEM),
        scratch_shapes=[
            pltpu.VMEM((N_DEV, B, SQ, D_QK), jnp.float32),
            pltpu.VMEM((N_DEV, B, SQ, 2 * HQ), jnp.float32),
            pltpu.SemaphoreType.DMA((N_DEV,)),
            pltpu.SemaphoreType.DMA((N_DEV,)),
            pltpu.SemaphoreType.DMA((N_DEV,)),
            pltpu.SemaphoreType.DMA((N_DEV,)),
        ],
        compiler_params=pltpu.CompilerParams(collective_id=0),
    )(x, Wq, k_shard, v_shard, Wo)


# device time: 21103 ns/iter; 3.0662x vs baseline; 1.5973x over previous
import jax
import jax.numpy as jnp
from jax import lax
from jax.experimental import pallas as pl
from jax.experimental.pallas import tpu as pltpu

N_DEV = 8
B = 2
SQ = 128
SKV = 1024
HQ = 4
DH = 64
D_MODEL = 512
D_QK = HQ * DH
KV_PER = SKV // N_DEV
RQ = SQ // N_DEV
NEG = -1e9


def kernel(x, Wq, K_ext, V_ext, Wo):
    k_shard = K_ext.reshape(B, KV_PER, D_QK)
    v_shard = V_ext.reshape(B, KV_PER, D_QK)

    def body(x_ref, wq_ref, k_ref, v_ref, wo_ref, out_ref,
             upart, spart, ucomb, scomb, oall,
             usend, urecv, ssend, srecv, osend, orecv):
        my = lax.axis_index("i")

        barrier_sem = pltpu.get_barrier_semaphore()
        for nbr in range(N_DEV):
            @pl.when(nbr != my)
            def _():
                pl.semaphore_signal(
                    barrier_sem, inc=1,
                    device_id=(nbr,), device_id_type=pl.DeviceIdType.MESH,
                )
        pl.semaphore_wait(barrier_sem, N_DEV - 1)

        qi = lax.broadcasted_iota(jnp.int32, (SQ, KV_PER), 0)
        kj = lax.broadcasted_iota(jnp.int32, (SQ, KV_PER), 1) + my * KV_PER
        mask = (jnp.abs(qi - kj) <= 128) | (kj < 32) | (qi < 32)

        for b in range(B):
            q_b = jnp.dot(x_ref[b], wq_ref[...],
                          preferred_element_type=jnp.float32)
            ms, ls = [], []
            for hh in range(HQ):
                sl = slice(hh * DH, (hh + 1) * DH)
                scores = lax.dot_general(
                    q_b[:, sl], k_ref[b][:, sl],
                    (((1,), (1,)), ((), ())),
                    preferred_element_type=jnp.float32,
                )
                s = jnp.where(mask, scores * 0.125, NEG)
                m = jnp.max(s, axis=1, keepdims=True)
                p = jnp.exp(s - m)
                l = jnp.sum(p, axis=1, keepdims=True)
                u = jnp.dot(p, v_ref[b][:, sl],
                            preferred_element_type=jnp.float32)
                for d in range(N_DEV):
                    upart[d, b, :, sl] = u[d * RQ:(d + 1) * RQ]
                ms.append(m)
                ls.append(l)
            st = jnp.concatenate(ms + ls, axis=1)
            for d in range(N_DEV):
                spart[d, b] = st[d * RQ:(d + 1) * RQ]

        for peer in range(N_DEV):
            @pl.when(peer != my)
            def _():
                pltpu.make_async_remote_copy(
                    src_ref=upart.at[peer], dst_ref=ucomb.at[my],
                    send_sem=usend.at[peer], recv_sem=urecv.at[my],
                    device_id=(peer,), device_id_type=pl.DeviceIdType.MESH,
                ).start()
                pltpu.make_async_remote_copy(
                    src_ref=spart.at[peer], dst_ref=scomb.at[my],
                    send_sem=ssend.at[peer], recv_sem=srecv.at[my],
                    device_id=(peer,), device_id_type=pl.DeviceIdType.MESH,
                ).start()
        ucomb[pl.ds(my, 1)] = upart[my][None]
        scomb[pl.ds(my, 1)] = spart[my][None]

        for o in range(N_DEV):
            @pl.when(o != my)
            def _():
                pltpu.make_async_remote_copy(
                    src_ref=ucomb.at[o], dst_ref=ucomb.at[o],
                    send_sem=usend.at[o], recv_sem=urecv.at[o],
                    device_id=(my,), device_id_type=pl.DeviceIdType.MESH,
                ).wait_recv()
                pltpu.make_async_remote_copy(
                    src_ref=scomb.at[o], dst_ref=scomb.at[o],
                    send_sem=ssend.at[o], recv_sem=srecv.at[o],
                    device_id=(my,), device_id_type=pl.DeviceIdType.MESH,
                ).wait_recv()

        for b in range(B):
            ctx_heads = []
            for hh in range(HQ):
                sl = slice(hh * DH, (hh + 1) * DH)
                ms = [scomb[o, b][:, hh:hh + 1] for o in range(N_DEV)]
                ls = [scomb[o, b][:, HQ + hh:HQ + hh + 1] for o in range(N_DEV)]
                M = ms[0]
                for o in range(1, N_DEV):
                    M = jnp.maximum(M, ms[o])
                ws = [jnp.exp(ms[o] - M) for o in range(N_DEV)]
                L = ws[0] * ls[0]
                acc = ws[0] * ucomb[0, b][:, sl]
                for o in range(1, N_DEV):
                    L = L + ws[o] * ls[o]
                    acc = acc + ws[o] * ucomb[o, b][:, sl]
                ctx_heads.append(acc / L)
            ctx = jnp.concatenate(ctx_heads, axis=1)
            o_b = jnp.dot(ctx, wo_ref[...],
                          preferred_element_type=jnp.float32)
            oall[pl.ds(my, 1), b] = o_b[None]

        for peer in range(N_DEV):
            @pl.when(peer != my)
            def _():
                pltpu.make_async_remote_copy(
                    src_ref=oall.at[my], dst_ref=oall.at[my],
                    send_sem=osend.at[peer], recv_sem=orecv.at[my],
                    device_id=(peer,), device_id_type=pl.DeviceIdType.MESH,
                ).start()
        for o in range(N_DEV):
            @pl.when(o != my)
            def _():
                pltpu.make_async_remote_copy(
                    src_ref=oall.at[o], dst_ref=oall.at[o],
                    send_sem=osend.at[o], recv_sem=orecv.at[o],
                    device_id=(my,), device_id_type=pl.DeviceIdType.MESH,
                ).wait_recv()
        for o in range(N_DEV):
            out_ref[:, o * RQ:(o + 1) * RQ, :] = oall[o]

        for peer in range(N_DEV):
            @pl.when(peer != my)
            def _():
                pltpu.make_async_remote_copy(
                    src_ref=upart.at[peer], dst_ref=ucomb.at[my],
                    send_sem=usend.at[peer], recv_sem=urecv.at[my],
                    device_id=(peer,), device_id_type=pl.DeviceIdType.MESH,
                ).wait_send()
                pltpu.make_async_remote_copy(
                    src_ref=spart.at[peer], dst_ref=scomb.at[my],
                    send_sem=ssend.at[peer], recv_sem=srecv.at[my],
                    device_id=(peer,), device_id_type=pl.DeviceIdType.MESH,
                ).wait_send()
                pltpu.make_async_remote_copy(
                    src_ref=oall.at[my], dst_ref=oall.at[my],
                    send_sem=osend.at[peer], recv_sem=orecv.at[my],
                    device_id=(peer,), device_id_type=pl.DeviceIdType.MESH,
                ).wait_send()

    return pl.pallas_call(
        body,
        out_shape=jax.ShapeDtypeStruct((B, SQ, D_MODEL), jnp.float32),
        in_specs=[pl.BlockSpec(memory_space=pltpu.VMEM)] * 5,
        out_specs=pl.BlockSpec(memory_space=pltpu.VMEM),
        scratch_shapes=[
            pltpu.VMEM((N_DEV, B, RQ, D_QK), jnp.float32),
            pltpu.VMEM((N_DEV, B, RQ, 2 * HQ), jnp.float32),
            pltpu.VMEM((N_DEV, B, RQ, D_QK), jnp.float32),
            pltpu.VMEM((N_DEV, B, RQ, 2 * HQ), jnp.float32),
            pltpu.VMEM((N_DEV, B, RQ, D_MODEL), jnp.float32),
            pltpu.SemaphoreType.DMA((N_DEV,)),
            pltpu.SemaphoreType.DMA((N_DEV,)),
            pltpu.SemaphoreType.DMA((N_DEV,)),
            pltpu.SemaphoreType.DMA((N_DEV,)),
            pltpu.SemaphoreType.DMA((N_DEV,)),
            pltpu.SemaphoreType.DMA((N_DEV,)),
        ],
        compiler_params=pltpu.CompilerParams(collective_id=0),
    )(x, Wq, k_shard, v_shard, Wo)


# device time: 20171 ns/iter; 3.2079x vs baseline; 1.0462x over previous
import jax
import jax.numpy as jnp
from jax import lax
from jax.experimental import pallas as pl
from jax.experimental.pallas import tpu as pltpu

N_DEV = 8
B = 2
SQ = 128
SKV = 1024
HQ = 4
DH = 64
D_MODEL = 512
D_QK = HQ * DH
KV_PER = SKV // N_DEV
RQ = SQ // N_DEV
NEG = -1e9


def kernel(x, Wq, K_ext, V_ext, Wo):
    def body(x_ref, wq_ref, k_ref, v_ref, wo_ref, out_ref,
             upart, spart, ucomb, scomb, omine,
             usend, urecv, ssend, srecv, osend, orecv):
        my = lax.axis_index("i")

        barrier_sem = pltpu.get_barrier_semaphore()
        for nbr in range(N_DEV):
            @pl.when(nbr != my)
            def _():
                pl.semaphore_signal(
                    barrier_sem, inc=1,
                    device_id=(nbr,), device_id_type=pl.DeviceIdType.MESH,
                )
        pl.semaphore_wait(barrier_sem, N_DEV - 1)

        qi = lax.broadcasted_iota(jnp.int32, (SQ, KV_PER), 0)
        kj = lax.broadcasted_iota(jnp.int32, (SQ, KV_PER), 1) + my * KV_PER
        mask = (jnp.abs(qi - kj) <= 128) | (kj < 32) | (qi < 32)

        for b in range(B):
            q_b = jnp.dot(x_ref[b], wq_ref[...],
                          preferred_element_type=jnp.float32)
            ms, ls = [], []
            for hh in range(HQ):
                sl = slice(hh * DH, (hh + 1) * DH)
                scores = lax.dot_general(
                    q_b[:, sl], k_ref[b, :, hh, :],
                    (((1,), (1,)), ((), ())),
                    preferred_element_type=jnp.float32,
                )
                s = jnp.where(mask, scores * 0.125, NEG)
                m = jnp.max(s, axis=1, keepdims=True)
                p = jnp.exp(s - m)
                l = jnp.sum(p, axis=1, keepdims=True)
                u = jnp.dot(p, v_ref[b, :, hh, :],
                            preferred_element_type=jnp.float32)
                for d in range(N_DEV):
                    upart[d, b, :, sl] = u[d * RQ:(d + 1) * RQ]
                ms.append(m)
                ls.append(l)
            st = jnp.concatenate(ms + ls, axis=1)
            for d in range(N_DEV):
                spart[d, b] = st[d * RQ:(d + 1) * RQ]

        for peer in range(N_DEV):
            @pl.when(peer != my)
            def _():
                pltpu.make_async_remote_copy(
                    src_ref=upart.at[peer], dst_ref=ucomb.at[my],
                    send_sem=usend.at[peer], recv_sem=urecv.at[my],
                    device_id=(peer,), device_id_type=pl.DeviceIdType.MESH,
                ).start()
                pltpu.make_async_remote_copy(
                    src_ref=spart.at[peer], dst_ref=scomb.at[my],
                    send_sem=ssend.at[peer], recv_sem=srecv.at[my],
                    device_id=(peer,), device_id_type=pl.DeviceIdType.MESH,
                ).start()
        ucomb[pl.ds(my, 1)] = upart[my][None]
        scomb[pl.ds(my, 1)] = spart[my][None]

        for o in range(N_DEV):
            @pl.when(o != my)
            def _():
                pltpu.make_async_remote_copy(
                    src_ref=ucomb.at[o], dst_ref=ucomb.at[o],
                    send_sem=usend.at[o], recv_sem=urecv.at[o],
                    device_id=(my,), device_id_type=pl.DeviceIdType.MESH,
                ).wait_recv()
                pltpu.make_async_remote_copy(
                    src_ref=scomb.at[o], dst_ref=scomb.at[o],
                    send_sem=ssend.at[o], recv_sem=srecv.at[o],
                    device_id=(my,), device_id_type=pl.DeviceIdType.MESH,
                ).wait_recv()

        for b in range(B):
            ctx_heads = []
            for hh in range(HQ):
                sl = slice(hh * DH, (hh + 1) * DH)
                ms = [scomb[o, b][:, hh:hh + 1] for o in range(N_DEV)]
                ls = [scomb[o, b][:, HQ + hh:HQ + hh + 1] for o in range(N_DEV)]
                M = ms[0]
                for o in range(1, N_DEV):
                    M = jnp.maximum(M, ms[o])
                ws = [jnp.exp(ms[o] - M) for o in range(N_DEV)]
                L = ws[0] * ls[0]
                acc = ws[0] * ucomb[0, b][:, sl]
                for o in range(1, N_DEV):
                    L = L + ws[o] * ls[o]
                    acc = acc + ws[o] * ucomb[o, b][:, sl]
                ctx_heads.append(acc / L)
            ctx = jnp.concatenate(ctx_heads, axis=1)
            o_b = jnp.dot(ctx, wo_ref[...],
                          preferred_element_type=jnp.float32)
            omine[b] = o_b
            out_ref[b, pl.ds(my * RQ, RQ), :] = o_b

        for peer in range(N_DEV):
            @pl.when(peer != my)
            def _():
                pltpu.make_async_remote_copy(
                    src_ref=omine, dst_ref=out_ref.at[:, pl.ds(my * RQ, RQ)],
                    send_sem=osend.at[peer], recv_sem=orecv.at[my],
                    device_id=(peer,), device_id_type=pl.DeviceIdType.MESH,
                ).start()
        for o in range(N_DEV):
            @pl.when(o != my)
            def _():
                pltpu.make_async_remote_copy(
                    src_ref=omine, dst_ref=out_ref.at[:, pl.ds(o * RQ, RQ)],
                    send_sem=osend.at[o], recv_sem=orecv.at[o],
                    device_id=(my,), device_id_type=pl.DeviceIdType.MESH,
                ).wait_recv()

        for peer in range(N_DEV):
            @pl.when(peer != my)
            def _():
                pltpu.make_async_remote_copy(
                    src_ref=upart.at[peer], dst_ref=ucomb.at[my],
                    send_sem=usend.at[peer], recv_sem=urecv.at[my],
                    device_id=(peer,), device_id_type=pl.DeviceIdType.MESH,
                ).wait_send()
                pltpu.make_async_remote_copy(
                    src_ref=spart.at[peer], dst_ref=scomb.at[my],
                    send_sem=ssend.at[peer], recv_sem=srecv.at[my],
                    device_id=(peer,), device_id_type=pl.DeviceIdType.MESH,
                ).wait_send()
                pltpu.make_async_remote_copy(
                    src_ref=omine, dst_ref=out_ref.at[:, pl.ds(my * RQ, RQ)],
                    send_sem=osend.at[peer], recv_sem=orecv.at[my],
                    device_id=(peer,), device_id_type=pl.DeviceIdType.MESH,
                ).wait_send()

    return pl.pallas_call(
        body,
        out_shape=jax.ShapeDtypeStruct((B, SQ, D_MODEL), jnp.float32),
        in_specs=[pl.BlockSpec(memory_space=pltpu.VMEM)] * 5,
        out_specs=pl.BlockSpec(memory_space=pltpu.VMEM),
        scratch_shapes=[
            pltpu.VMEM((N_DEV, B, RQ, D_QK), jnp.float32),
            pltpu.VMEM((N_DEV, B, RQ, 2 * HQ), jnp.float32),
            pltpu.VMEM((N_DEV, B, RQ, D_QK), jnp.float32),
            pltpu.VMEM((N_DEV, B, RQ, 2 * HQ), jnp.float32),
            pltpu.VMEM((B, RQ, D_MODEL), jnp.float32),
            pltpu.SemaphoreType.DMA((N_DEV,)),
            pltpu.SemaphoreType.DMA((N_DEV,)),
            pltpu.SemaphoreType.DMA((N_DEV,)),
            pltpu.SemaphoreType.DMA((N_DEV,)),
            pltpu.SemaphoreType.DMA((N_DEV,)),
            pltpu.SemaphoreType.DMA((N_DEV,)),
        ],
        compiler_params=pltpu.CompilerParams(collective_id=0),
    )(x, Wq, K_ext, V_ext, Wo)
